# baseline (speedup 1.0000x reference)
import functools

import jax
import jax.numpy as jnp
from jax import lax
from jax.experimental import pallas as pl
from jax.experimental.pallas import tpu as pltpu
from jax.experimental.pallas import tpu_sc as plsc

E = 160000
C = 128
NSEG = 2
ES = E // NSEG
NC, NS = 2, 16
NW = NC * NS
CH = 32
NCHUNKS = ES // CH
NOCT = -(-(-(-NCHUNKS // NW)) // 8)
TE = 640


def _sc_gather(xt, idx):
    mesh = plsc.VectorSubcoreMesh(
        core_axis_name="c", subcore_axis_name="s",
        num_cores=NC, num_subcores=NS)

    @functools.partial(
        pl.kernel,
        out_type=jax.ShapeDtypeStruct((4, ES, C), jnp.float32),
        mesh=mesh,
        scratch_types=(
            [pltpu.VMEM((4 * CH,), jnp.int32) for _ in range(8)]
            + [pltpu.VMEM((CH, C), jnp.float32) for _ in range(16)]
            + [pltpu.SemaphoreType.DMA for _ in range(16)]
        ),
        compiler_params=pltpu.CompilerParams(needs_layout_passes=False),
    )
    def gather_kernel(xt_hbm, idx_hbm, g_hbm, *scr):
        idxb = scr[0:8]
        fin = tuple(tuple(scr[8 + 4 * s + k] for k in range(4))
                    for s in range(4))
        gsem = scr[24:28]
        wsem = scr[28:32]
        isem = scr[32:40]
        wid = lax.axis_index("s") * NC + lax.axis_index("c")

        def chunk_of(j):
            return wid + j * NW

        for s in range(4):
            @pl.when(chunk_of(s) < NCHUNKS)
            def _(s=s):
                pltpu.async_copy(idx_hbm.at[chunk_of(s)], idxb[s], isem[s])

        def oct_body(oo, carry):
            for half in range(2):
                for s in range(4):
                    j = 8 * oo + 4 * half + s
                    a = 4 * half + s
                    chunk = chunk_of(j)

                    @pl.when(chunk < NCHUNKS)
                    def _(s=s, a=a, j=j, chunk=chunk):
                        pltpu.make_async_copy(
                            idx_hbm.at[0], idxb[a], isem[a]).wait()
                        @pl.when(j >= 4)
                        def _():
                            for k in range(4):
                                pltpu.make_async_copy(
                                    fin[s][k], g_hbm.at[k, pl.ds(0, CH)],
                                    wsem[s]).wait()
                        for k in range(4):
                            pltpu.async_copy(
                                xt_hbm.at[idxb[a].at[pl.ds(CH * k, CH)]],
                                fin[s][k], gsem[s])

                for s in range(4):
                    jn = 8 * oo + 4 * (half + 1) + s
                    a2 = 4 * ((half + 1) % 2) + s

                    @pl.when(chunk_of(jn) < NCHUNKS)
                    def _(s=s, a2=a2, jn=jn):
                        pltpu.async_copy(idx_hbm.at[chunk_of(jn)],
                                         idxb[a2], isem[a2])

                for s in range(4):
                    j = 8 * oo + 4 * half + s
                    a = 4 * half + s
                    chunk = chunk_of(j)

                    @pl.when(chunk < NCHUNKS)
                    def _(s=s, a=a, chunk=chunk):
                        for k in range(4):
                            pltpu.make_async_copy(
                                xt_hbm.at[idxb[a].at[pl.ds(CH * k, CH)]],
                                fin[s][k], gsem[s]).wait()
                        base = chunk * CH
                        for k in range(4):
                            pltpu.async_copy(fin[s][k],
                                             g_hbm.at[k, pl.ds(base, CH)],
                                             wsem[s])

            return carry

        lax.fori_loop(0, NOCT, oct_body, 0)

        for s in range(4):
            @pl.when(chunk_of(s) < NCHUNKS)
            def _(s=s):
                for k in range(4):
                    pltpu.make_async_copy(
                        fin[s][k], g_hbm.at[k, pl.ds(0, CH)],
                        wsem[s]).wait()

    return gather_kernel(xt, idx)


def _tc_body(xt_ref, g_ref, wt_ref, b_ref, out_ref):
    y1 = g_ref[0]
    y2 = g_ref[1]
    y3 = g_ref[2]
    y4 = g_ref[3]
    s1 = y1 + y3
    s2 = y2 + y4
    d1 = jnp.abs(y1 - y3)
    d2 = jnp.abs(y2 - y4)
    dn = (((1,), (1,)), ((), ()))
    acc = lax.dot_general(wt_ref[0], xt_ref[...], dn,
                          preferred_element_type=jnp.float32)
    acc = acc + lax.dot_general(wt_ref[1], s1, dn,
                                preferred_element_type=jnp.float32)
    acc = acc + lax.dot_general(wt_ref[2], s2, dn,
                                preferred_element_type=jnp.float32)
    acc = acc + lax.dot_general(wt_ref[3], d1, dn,
                                preferred_element_type=jnp.float32)
    acc = acc + lax.dot_general(wt_ref[4], d2, dn,
                                preferred_element_type=jnp.float32)
    out_ref[...] = acc + b_ref[...]


def _tc_conv(seg, xt_full, g, wt, b_col, prev_out):
    col0 = seg * (ES // TE)
    kwargs = {}
    args = [xt_full, g, wt, b_col]
    in_specs = [
        pl.BlockSpec((TE, C), lambda i, col0=col0: (col0 + i, 0)),
        pl.BlockSpec((4, TE, C), lambda i: (0, i, 0)),
        pl.BlockSpec((5, C, C), lambda i: (0, 0, 0)),
        pl.BlockSpec((C, 1), lambda i: (0, 0)),
    ]
    if prev_out is not None:
        args.append(prev_out)
        in_specs.append(pl.BlockSpec(memory_space=pl.ANY))
        kwargs["input_output_aliases"] = {4: 0}

    def body(*refs):
        _tc_body(*refs[:4], refs[-1])

    return pl.pallas_call(
        body,
        grid=(ES // TE,),
        in_specs=in_specs,
        out_specs=pl.BlockSpec((C, TE), lambda i, col0=col0: (0, col0 + i)),
        out_shape=jax.ShapeDtypeStruct((C, E), jnp.float32),
        **kwargs,
    )(*args)


def kernel(x, gemm_edges, W, b):
    xt = x[0, :, :, 0].T
    idx = (gemm_edges[0].astype(jnp.int32)
           .reshape(NSEG, NCHUNKS, CH, 4).transpose(0, 1, 3, 2)
           .reshape(NSEG, NCHUNKS, 4 * CH))
    wt = W[:, :, 0, :].transpose(2, 0, 1)
    b_col = b[:, None]
    gs = [_sc_gather(xt, idx[s]) for s in range(NSEG)]
    out = None
    for s in range(NSEG):
        out = _tc_conv(s, xt, gs[s], wt, b_col, out)
    return out[None, :, :, None]


# speedup vs baseline: 1.1627x; 1.1627x over previous
import functools

import jax
import jax.numpy as jnp
from jax import lax
from jax.experimental import pallas as pl
from jax.experimental.pallas import tpu as pltpu
from jax.experimental.pallas import tpu_sc as plsc

E = 160000
C = 128
NSEG = 2
ES = E // NSEG
NC, NS = 2, 16
NW = NC * NS
CH = 32
NCHUNKS = ES // CH
NOCT = -(-(-(-NCHUNKS // NW)) // 8)
TE = 3200


def _sc_gather(xt, idx):
    mesh = plsc.VectorSubcoreMesh(
        core_axis_name="c", subcore_axis_name="s",
        num_cores=NC, num_subcores=NS)

    @functools.partial(
        pl.kernel,
        out_type=jax.ShapeDtypeStruct((4, ES, C), jnp.float32),
        mesh=mesh,
        scratch_types=(
            [pltpu.VMEM((4 * CH,), jnp.int32) for _ in range(8)]
            + [pltpu.VMEM((CH, C), jnp.float32) for _ in range(16)]
            + [pltpu.SemaphoreType.DMA for _ in range(16)]
        ),
        compiler_params=pltpu.CompilerParams(needs_layout_passes=False),
    )
    def gather_kernel(xt_hbm, idx_hbm, g_hbm, *scr):
        idxb = scr[0:8]
        fin = tuple(tuple(scr[8 + 4 * s + k] for k in range(4))
                    for s in range(4))
        gsem = scr[24:28]
        wsem = scr[28:32]
        isem = scr[32:40]
        wid = lax.axis_index("s") * NC + lax.axis_index("c")

        def chunk_of(j):
            return wid + j * NW

        for s in range(4):
            @pl.when(chunk_of(s) < NCHUNKS)
            def _(s=s):
                pltpu.async_copy(idx_hbm.at[chunk_of(s)], idxb[s], isem[s])

        def oct_body(oo, carry):
            for half in range(2):
                for s in range(4):
                    j = 8 * oo + 4 * half + s
                    a = 4 * half + s
                    chunk = chunk_of(j)

                    @pl.when(chunk < NCHUNKS)
                    def _(s=s, a=a, j=j, chunk=chunk):
                        pltpu.make_async_copy(
                            idx_hbm.at[0], idxb[a], isem[a]).wait()
                        @pl.when(j >= 4)
                        def _():
                            for k in range(4):
                                pltpu.make_async_copy(
                                    fin[s][k], g_hbm.at[k, pl.ds(0, CH)],
                                    wsem[s]).wait()
                        for k in range(4):
                            pltpu.async_copy(
                                xt_hbm.at[idxb[a].at[pl.ds(CH * k, CH)]],
                                fin[s][k], gsem[s])

                for s in range(4):
                    jn = 8 * oo + 4 * (half + 1) + s
                    a2 = 4 * ((half + 1) % 2) + s

                    @pl.when(chunk_of(jn) < NCHUNKS)
                    def _(s=s, a2=a2, jn=jn):
                        pltpu.async_copy(idx_hbm.at[chunk_of(jn)],
                                         idxb[a2], isem[a2])

                for s in range(4):
                    j = 8 * oo + 4 * half + s
                    a = 4 * half + s
                    chunk = chunk_of(j)

                    @pl.when(chunk < NCHUNKS)
                    def _(s=s, a=a, chunk=chunk):
                        for k in range(4):
                            pltpu.make_async_copy(
                                xt_hbm.at[idxb[a].at[pl.ds(CH * k, CH)]],
                                fin[s][k], gsem[s]).wait()
                        base = chunk * CH
                        for k in range(4):
                            pltpu.async_copy(fin[s][k],
                                             g_hbm.at[k, pl.ds(base, CH)],
                                             wsem[s])

            return carry

        lax.fori_loop(0, NOCT, oct_body, 0)

        for s in range(4):
            @pl.when(chunk_of(s) < NCHUNKS)
            def _(s=s):
                for k in range(4):
                    pltpu.make_async_copy(
                        fin[s][k], g_hbm.at[k, pl.ds(0, CH)],
                        wsem[s]).wait()

    return gather_kernel(xt, idx)


def _tc_body(xt_ref, g_ref, wt_ref, b_ref, out_ref):
    y1 = g_ref[0]
    y2 = g_ref[1]
    y3 = g_ref[2]
    y4 = g_ref[3]
    s1 = y1 + y3
    s2 = y2 + y4
    d1 = jnp.abs(y1 - y3)
    d2 = jnp.abs(y2 - y4)
    dn = (((1,), (1,)), ((), ()))
    acc = lax.dot_general(wt_ref[0], xt_ref[...], dn,
                          preferred_element_type=jnp.float32)
    acc = acc + lax.dot_general(wt_ref[1], s1, dn,
                                preferred_element_type=jnp.float32)
    acc = acc + lax.dot_general(wt_ref[2], s2, dn,
                                preferred_element_type=jnp.float32)
    acc = acc + lax.dot_general(wt_ref[3], d1, dn,
                                preferred_element_type=jnp.float32)
    acc = acc + lax.dot_general(wt_ref[4], d2, dn,
                                preferred_element_type=jnp.float32)
    out_ref[...] = acc + b_ref[...]


def _tc_conv(seg, xt_full, g, wt, b_col, prev_out):
    col0 = seg * (ES // TE)
    kwargs = {}
    args = [xt_full, g, wt, b_col]
    in_specs = [
        pl.BlockSpec((TE, C), lambda i, col0=col0: (col0 + i, 0)),
        pl.BlockSpec((4, TE, C), lambda i: (0, i, 0)),
        pl.BlockSpec((5, C, C), lambda i: (0, 0, 0)),
        pl.BlockSpec((C, 1), lambda i: (0, 0)),
    ]
    if prev_out is not None:
        args.append(prev_out)
        in_specs.append(pl.BlockSpec(memory_space=pl.ANY))
        kwargs["input_output_aliases"] = {4: 0}

    def body(*refs):
        _tc_body(*refs[:4], refs[-1])

    return pl.pallas_call(
        body,
        grid=(ES // TE,),
        in_specs=in_specs,
        out_specs=pl.BlockSpec((C, TE), lambda i, col0=col0: (0, col0 + i)),
        out_shape=jax.ShapeDtypeStruct((C, E), jnp.float32),
        **kwargs,
    )(*args)


def kernel(x, gemm_edges, W, b):
    xt = x[0, :, :, 0].T
    idx = (gemm_edges[0].astype(jnp.int32)
           .reshape(NSEG, NCHUNKS, CH, 4).transpose(0, 1, 3, 2)
           .reshape(NSEG, NCHUNKS, 4 * CH))
    wt = W[:, :, 0, :].transpose(2, 0, 1)
    b_col = b[:, None]
    gs = [_sc_gather(xt, idx[s]) for s in range(NSEG)]
    out = None
    for s in range(NSEG):
        out = _tc_conv(s, xt, gs[s], wt, b_col, out)
    return out[None, :, :, None]
